# baseline (speedup 1.0000x reference)
import functools

import jax
import jax.numpy as jnp
from jax import lax
from jax.experimental import pallas as pl
from jax.experimental.pallas import tpu as pltpu
from jax.experimental.pallas import tpu_sc as plsc

_NC, _NS, _L = 2, 16, 16
_NW = _NC * _NS


@functools.partial(jax.jit, static_argnames=("n", "n_values", "chunk"))
def _lookup_flat(values, idx_flat, *, n, n_values, chunk):
    per_w = n // _NW
    nchunk = per_w // chunk
    mesh = plsc.VectorSubcoreMesh(core_axis_name="c", subcore_axis_name="s")

    @functools.partial(
        pl.kernel,
        out_type=jax.ShapeDtypeStruct((n,), jnp.float32),
        mesh=mesh,
        compiler_params=pltpu.CompilerParams(needs_layout_passes=False),
        scratch_types=[
            pltpu.VMEM((128,), jnp.float32),
            pltpu.VMEM((chunk,), jnp.int32),
            pltpu.VMEM((chunk,), jnp.float32),
        ],
    )
    def k(values_hbm, idx_hbm, out_hbm, tbl, idx_v, out_v):
        wid = lax.axis_index("s") * _NC + lax.axis_index("c")
        base = wid * per_w
        pltpu.sync_copy(values_hbm, tbl.at[pl.ds(0, n_values)])
        for c in range(nchunk):
            off = base + c * chunk
            pltpu.sync_copy(idx_hbm.at[pl.ds(off, chunk)], idx_v)

            @plsc.parallel_loop(0, chunk, step=_L, unroll=8)
            def _(i):
                iv = idx_v[pl.ds(i, _L)]
                out_v[pl.ds(i, _L)] = plsc.load_gather(tbl, [iv])

            pltpu.sync_copy(out_v, out_hbm.at[pl.ds(off, chunk)])

    return k(values, idx_flat)


def kernel(values, index):
    n_structure, n_atoms = index.shape
    n = n_structure * n_atoms
    out = _lookup_flat(
        values,
        index.reshape(n),
        n=n,
        n_values=values.shape[0],
        chunk=12800,
    )
    return out.reshape(n_structure, n_atoms)
